# Initial kernel scaffold; baseline (speedup 1.0000x reference)
import jax
import jax.numpy as jnp
from jax.experimental import pallas as pl


def kernel(node_features, edge_index, node_type, per_type_std, per_type_bias):
    raise NotImplementedError("write your pallas kernel here")


# speedup vs baseline: 55.2708x; 55.2708x over previous
import functools

import jax
import jax.numpy as jnp
from jax import lax
from jax.experimental import pallas as pl
from jax.experimental.pallas import tpu as pltpu
from jax.experimental.pallas import tpu_sc as plsc

_N = 100000
_E = 6400000
_T = 16

_NC, _NS = 2, 16
_NW = _NC * _NS
_NPAD = 100096
_ROWS_P2 = _NPAD // 128
_PER_TILE = _NPAD // _NS
_EPW = _E // _NW
_CHUNK = 1280
_NCHUNK = -(-_EPW // _CHUNK)

@functools.cache
def _build_phase1():
    mesh = plsc.VectorSubcoreMesh(
        core_axis_name="c", subcore_axis_name="s", num_cores=_NC, num_subcores=_NS
    )
    return functools.partial(
        pl.kernel,
        out_type=jax.ShapeDtypeStruct((_NC * _NPAD,), jnp.int32),
        mesh=mesh,
        scratch_types=[
            pltpu.VMEM((_PER_TILE,), jnp.int32),
            pltpu.VMEM((_CHUNK,), jnp.int32),
            pltpu.VMEM((_CHUNK,), jnp.int32),
            pltpu.VMEM_SHARED((_NPAD,), jnp.int32),
        ],
    )(_phase1_body)


def _phase1_body(edge_hbm, out_hbm, stage_v, idx_v, ones_v, counts_sh):
    c = lax.axis_index("c")
    s = lax.axis_index("s")
    wid = s * _NC + c

    def _zero(i, carry):
        stage_v[pl.ds(i * 16, 16)] = jnp.zeros((16,), jnp.int32)
        return carry

    lax.fori_loop(jnp.int32(0), jnp.int32(_PER_TILE // 16), _zero, 0)
    pltpu.sync_copy(stage_v, counts_sh.at[pl.ds(s * _PER_TILE, _PER_TILE)])

    def _one(i, carry):
        ones_v[pl.ds(i * 16, 16)] = jnp.ones((16,), jnp.int32)
        return carry

    lax.fori_loop(jnp.int32(0), jnp.int32(_CHUNK // 16), _one, 0)
    plsc.subcore_barrier()

    wstart = wid * _EPW

    def _chunk(k, carry):
        base = jnp.minimum(wstart + k * _CHUNK, _E - _CHUNK)
        pltpu.sync_copy(edge_hbm.at[pl.ds(base, _CHUNK)], idx_v)
        pltpu.sync_copy(ones_v, counts_sh.at[idx_v], add=True)
        return carry

    lax.fori_loop(jnp.int32(0), jnp.int32(_NCHUNK), _chunk, 0)
    plsc.subcore_barrier()

    pltpu.sync_copy(counts_sh.at[pl.ds(s * _PER_TILE, _PER_TILE)], stage_v)
    pltpu.sync_copy(stage_v, out_hbm.at[pl.ds(c * _NPAD + s * _PER_TILE, _PER_TILE)])


def _phase2_body(f_ref, sp_ref, cnt_ref, std_ref, bias_ref, o_ref):
    f = f_ref[...]
    sp = sp_ref[...]
    center = (cnt_ref[0] + cnt_ref[1]) > 0
    sg = jnp.zeros_like(f)
    bg = jnp.zeros_like(f)
    for t in range(_T):
        m = sp == t
        sg = sg + jnp.where(m, std_ref[t], 0.0)
        bg = bg + jnp.where(m, bias_ref[t], 0.0)
    o_ref[...] = jnp.where(center, f * sg + bg, f)


def kernel(node_features, edge_index, node_type, per_type_std, per_type_bias):
    edge_src = edge_index[0].astype(jnp.int32)
    counts = _build_phase1()(edge_src)

    f_pad = jnp.pad(node_features[:, 0], (0, _NPAD - _N)).reshape(_ROWS_P2, 128)
    sp_pad = jnp.pad(node_type[:, 0].astype(jnp.int32), (0, _NPAD - _N)).reshape(
        _ROWS_P2, 128
    )
    cnt3 = counts.reshape(_NC, _ROWS_P2, 128)

    out2 = pl.pallas_call(
        _phase2_body,
        out_shape=jax.ShapeDtypeStruct((_ROWS_P2, 128), jnp.float32),
        in_specs=[
            pl.BlockSpec(memory_space=pltpu.VMEM),
            pl.BlockSpec(memory_space=pltpu.VMEM),
            pl.BlockSpec(memory_space=pltpu.VMEM),
            pl.BlockSpec(memory_space=pltpu.SMEM),
            pl.BlockSpec(memory_space=pltpu.SMEM),
        ],
    )(f_pad, sp_pad, cnt3, per_type_std[:, 0], per_type_bias[:, 0])

    return out2.reshape(_NPAD)[:_N].reshape(_N, 1)
